# Initial kernel scaffold; baseline (speedup 1.0000x reference)
import jax
import jax.numpy as jnp
from jax.experimental import pallas as pl


def kernel(x, edge_index, batch, covariates_embedding, W, b):
    raise NotImplementedError("write your pallas kernel here")


# speedup vs baseline: 2.0722x; 2.0722x over previous
import functools

import jax
import jax.numpy as jnp
from jax import lax
from jax.experimental import pallas as pl
from jax.experimental.pallas import tpu as pltpu, tpu_sc as plsc

N = 100000
H = 128
S = 1024
NC = 2
NS = 16
NW = NC * NS
CH = 3136
TILE = 448
NT = CH // TILE
GPT = TILE // 16
BATBUF = 3168


def _sc_body(x_hbm, batp_hbm, w1_hbm, py_hbm, pc_hbm,
             xb0, xb1, batbuf, w1buf, ysum, cnt, sem0, sem1):
    wid = lax.axis_index("s") * NC + lax.axis_index("c")
    row0_w = wid * CH
    valid_end = jnp.minimum(row0_w + CH, N)
    lane = lax.iota(jnp.int32, 16)

    def _zero(i, _):
        z = jnp.zeros((16,), jnp.float32)
        ysum[pl.ds(i * 16, 16)] = z
        cnt[pl.ds(i * 16, 16)] = z
        return 0

    lax.fori_loop(0, S // 16, _zero, 0)

    pltpu.sync_copy(w1_hbm, w1buf)
    pltpu.sync_copy(batp_hbm.at[pl.ds(row0_w, BATBUF)], batbuf)

    w1s = []
    for j in range(H // 16):
        v = w1buf[pl.ds(j * 16, 16)]
        for k in range(16):
            w1s.append(v[k])

    xbufs = (xb0, xb1)
    sems = (sem0, sem1)

    def _start_dma(t, buf, sem):
        row0 = row0_w + t * TILE
        dr = jnp.minimum(row0, N - TILE)
        return (pltpu.async_copy(x_hbm.at[pl.ds(dr * H, TILE * H)], buf, sem),
                row0 - dr)

    cur = _start_dma(0, xbufs[0], sems[0])
    carry_G = jnp.float32(0.0)
    for t in range(NT):
        nxt = _start_dma(t + 1, xbufs[(t + 1) % 2], sems[(t + 1) % 2]) \
            if t + 1 < NT else None
        (cp, shift) = cur
        cp.wait()
        xb = xbufs[t % 2]

        def _group(g, G, t=t, xb=xb, shift=shift):
            loc = shift + g * 16 + lane
            loc = jnp.minimum(loc, TILE - 1)
            glob = row0_w + t * TILE + g * 16 + lane

            flat = loc * H
            acc = jnp.zeros((16,), jnp.float32)
            for f in range(H):
                acc = acc + plsc.load_gather(xb, [flat + f]) * w1s[f]

            cs = G + jnp.cumsum(acc)

            bidx = t * TILE + g * 16 + 1 + lane
            b = plsc.load_gather(batbuf, [bidx])
            b_prev = plsc.load_gather(batbuf, [bidx - 1])
            b_next = plsc.load_gather(batbuf, [bidx + 1])

            vmask = glob < valid_end
            m_s = ((b != b_prev) | (glob == row0_w)) & vmask
            m_e = ((b != b_next) | (glob == valid_end - 1)) & vmask

            plsc.addupdate_scatter(ysum, [b], cs, mask=m_e)
            plsc.addupdate_scatter(ysum, [b], acc - cs, mask=m_s)
            gc = (glob - row0_w + 1).astype(jnp.float32)
            plsc.addupdate_scatter(cnt, [b], gc, mask=m_e)
            plsc.addupdate_scatter(cnt, [b], 1.0 - gc, mask=m_s)
            return G + jnp.sum(acc)

        carry_G = lax.fori_loop(0, GPT, _group, carry_G)
        cur = nxt

    pltpu.sync_copy(ysum, py_hbm.at[wid])
    pltpu.sync_copy(cnt, pc_hbm.at[wid])


_sc_partial = functools.partial(
    pl.kernel,
    out_type=(
        jax.ShapeDtypeStruct((NW, S), jnp.float32),
        jax.ShapeDtypeStruct((NW, S), jnp.float32),
    ),
    mesh=plsc.VectorSubcoreMesh(core_axis_name="c", subcore_axis_name="s",
                                num_cores=NC, num_subcores=NS),
    compiler_params=pltpu.CompilerParams(needs_layout_passes=False),
    scratch_types=[
        pltpu.VMEM((TILE * H,), jnp.float32),
        pltpu.VMEM((TILE * H,), jnp.float32),
        pltpu.VMEM((BATBUF,), jnp.int32),
        pltpu.VMEM((H,), jnp.float32),
        pltpu.VMEM((S,), jnp.float32),
        pltpu.VMEM((S,), jnp.float32),
        pltpu.SemaphoreType.DMA,
        pltpu.SemaphoreType.DMA,
    ],
)(_sc_body)


def _combine_body(py_ref, pc_ref, cov_ref, w2_ref, b_ref, out_ref):
    ys = jnp.sum(py_ref[...], axis=0)
    cn = jnp.sum(pc_ref[...], axis=0)
    mean = ys / jnp.maximum(cn, 1.0)
    covw = jnp.dot(cov_ref[...], w2_ref[...],
                   preferred_element_type=jnp.float32)
    out_ref[...] = mean[:, None] + covw + b_ref[0, 0]


_combine = pl.pallas_call(
    _combine_body,
    out_shape=jax.ShapeDtypeStruct((S, 1), jnp.float32),
)


def kernel(x, edge_index, batch, covariates_embedding, W, b):
    del edge_index
    batch32 = batch.astype(jnp.int32)
    pad = (NW - 1) * CH + BATBUF - (N + 1) + 16
    batp = jnp.concatenate(
        [jnp.zeros((1,), jnp.int32), batch32, jnp.zeros((pad,), jnp.int32)])
    w1 = W[0, :H]
    w2 = W[0, H:].reshape(H, 1)
    py, pc = _sc_partial(x.reshape(-1), batp, w1)
    return _combine(py, pc, covariates_embedding, w2, b.reshape(1, 1))


# speedup vs baseline: 5.6227x; 2.7133x over previous
import functools

import jax
import jax.numpy as jnp
from jax import lax
from jax.experimental import pallas as pl
from jax.experimental.pallas import tpu as pltpu, tpu_sc as plsc

N = 100000
H = 128
S = 1024
NC = 2
NS = 16
NW = NC * NS
CH = 3136
TILE = 448
NT = CH // TILE
GPT = TILE // 16
BATBUF = 3168


def _sc_body(x_hbm, batp_hbm, w1_hbm, py_hbm, pc_hbm,
             xb0, xb1, batbuf, w1buf, ysum, cnt, ybuf, sem0, sem1):
    wid = lax.axis_index("s") * NC + lax.axis_index("c")
    row0_w = wid * CH
    valid_end = jnp.minimum(row0_w + CH, N)
    lane = lax.iota(jnp.int32, 16)

    def _zero(i, _):
        z = jnp.zeros((16,), jnp.float32)
        ysum[pl.ds(i * 16, 16)] = z
        cnt[pl.ds(i * 16, 16)] = z
        return 0

    lax.fori_loop(0, S // 16, _zero, 0)

    pltpu.sync_copy(w1_hbm, w1buf)
    pltpu.sync_copy(batp_hbm.at[pl.ds(row0_w, BATBUF)], batbuf)

    w1v = [w1buf[pl.ds(k * 16, 16)] for k in range(H // 16)]
    l17 = lane * 17

    xbufs = (xb0, xb1)
    sems = (sem0, sem1)

    def _start_dma(t, buf, sem):
        row0 = row0_w + t * TILE
        dr = jnp.minimum(row0, N - TILE)
        return (pltpu.async_copy(x_hbm.at[pl.ds(dr * H, TILE * H)], buf, sem),
                row0 - dr)

    cur = _start_dma(0, xbufs[0], sems[0])
    carry_G = jnp.float32(0.0)
    for t in range(NT):
        nxt = _start_dma(t + 1, xbufs[(t + 1) % 2], sems[(t + 1) % 2]) \
            if t + 1 < NT else None
        (cp, shift) = cur
        cp.wait()
        xb = xbufs[t % 2]

        def _group(g, G, t=t, xb=xb, shift=shift):
            base = shift + g * 16
            glob = row0_w + t * TILE + g * 16 + lane

            for j in range(16):
                off = jnp.minimum(base + j, TILE - 1) * H
                tt = xb[pl.ds(off, 16)] * w1v[0]
                for k in range(1, H // 16):
                    tt = tt + xb[pl.ds(off + k * 16, 16)] * w1v[k]
                ybuf[pl.ds(j * 17, 16)] = tt
            acc = plsc.load_gather(ybuf, [l17])
            for c in range(1, 16):
                acc = acc + plsc.load_gather(ybuf, [l17 + c])

            cs = G + jnp.cumsum(acc)

            bidx = t * TILE + g * 16 + 1 + lane
            b = plsc.load_gather(batbuf, [bidx])
            b_prev = plsc.load_gather(batbuf, [bidx - 1])
            b_next = plsc.load_gather(batbuf, [bidx + 1])

            vmask = glob < valid_end
            m_s = ((b != b_prev) | (glob == row0_w)) & vmask
            m_e = ((b != b_next) | (glob == valid_end - 1)) & vmask

            plsc.addupdate_scatter(ysum, [b], cs, mask=m_e)
            plsc.addupdate_scatter(ysum, [b], acc - cs, mask=m_s)
            gc = (glob - row0_w + 1).astype(jnp.float32)
            plsc.addupdate_scatter(cnt, [b], gc, mask=m_e)
            plsc.addupdate_scatter(cnt, [b], 1.0 - gc, mask=m_s)
            return G + jnp.sum(acc)

        carry_G = lax.fori_loop(0, GPT, _group, carry_G)
        cur = nxt

    pltpu.sync_copy(ysum, py_hbm.at[wid])
    pltpu.sync_copy(cnt, pc_hbm.at[wid])


_sc_partial = functools.partial(
    pl.kernel,
    out_type=(
        jax.ShapeDtypeStruct((NW, S), jnp.float32),
        jax.ShapeDtypeStruct((NW, S), jnp.float32),
    ),
    mesh=plsc.VectorSubcoreMesh(core_axis_name="c", subcore_axis_name="s",
                                num_cores=NC, num_subcores=NS),
    compiler_params=pltpu.CompilerParams(needs_layout_passes=False),
    scratch_types=[
        pltpu.VMEM((TILE * H,), jnp.float32),
        pltpu.VMEM((TILE * H,), jnp.float32),
        pltpu.VMEM((BATBUF,), jnp.int32),
        pltpu.VMEM((H,), jnp.float32),
        pltpu.VMEM((S,), jnp.float32),
        pltpu.VMEM((S,), jnp.float32),
        pltpu.VMEM((16 * 17 + 16,), jnp.float32),
        pltpu.SemaphoreType.DMA,
        pltpu.SemaphoreType.DMA,
    ],
)(_sc_body)


def _combine_body(py_ref, pc_ref, cov_ref, w2_ref, b_ref, out_ref):
    ys = jnp.sum(py_ref[...], axis=0)
    cn = jnp.sum(pc_ref[...], axis=0)
    mean = ys / jnp.maximum(cn, 1.0)
    covw = jnp.dot(cov_ref[...], w2_ref[...],
                   preferred_element_type=jnp.float32)
    out_ref[...] = mean[:, None] + covw + b_ref[0, 0]


_combine = pl.pallas_call(
    _combine_body,
    out_shape=jax.ShapeDtypeStruct((S, 1), jnp.float32),
)


def kernel(x, edge_index, batch, covariates_embedding, W, b):
    del edge_index
    batch32 = batch.astype(jnp.int32)
    pad = (NW - 1) * CH + BATBUF - (N + 1) + 16
    batp = jnp.concatenate(
        [jnp.zeros((1,), jnp.int32), batch32, jnp.zeros((pad,), jnp.int32)])
    w1 = W[0, :H]
    w2 = W[0, H:].reshape(H, 1)
    py, pc = _sc_partial(x.reshape(-1), batp, w1)
    return _combine(py, pc, covariates_embedding, w2, b.reshape(1, 1))


# speedup vs baseline: 7.8529x; 1.3966x over previous
import functools

import jax
import jax.numpy as jnp
from jax import lax
from jax.experimental import pallas as pl
from jax.experimental.pallas import tpu as pltpu, tpu_sc as plsc

N = 100000
H = 128
S = 1024
NC = 2
NS = 16
NW = NC * NS
CH = 3136
TILE = 448
NT = CH // TILE
GPT = TILE // 16
BATBUF = 3168


def _sc_body(x_hbm, batp_hbm, w1_hbm, py_hbm, pc_hbm,
             xb0, xb1, batbuf, w1buf, ysum, cnt, ybuf, sem0, sem1):
    wid = lax.axis_index("s") * NC + lax.axis_index("c")
    row0_w = wid * CH
    valid_end = jnp.minimum(row0_w + CH, N)
    lane = lax.iota(jnp.int32, 16)

    def _zero(i, _):
        z = jnp.zeros((16,), jnp.float32)
        ysum[pl.ds(i * 16, 16)] = z
        cnt[pl.ds(i * 16, 16)] = z
        return 0

    lax.fori_loop(0, S // 16, _zero, 0)

    pltpu.sync_copy(w1_hbm, w1buf)
    pltpu.sync_copy(batp_hbm.at[pl.ds(row0_w, BATBUF)], batbuf)

    w1v = [w1buf[pl.ds(k * 16, 16)] for k in range(H // 16)]
    l17 = lane * 17

    xbufs = (xb0, xb1)
    sems = (sem0, sem1)

    def _start_dma(t, buf, sem):
        row0 = row0_w + t * TILE
        dr = jnp.minimum(row0, N - TILE)
        return (pltpu.async_copy(x_hbm.at[pl.ds(dr * H, TILE * H)], buf, sem),
                row0 - dr)

    cur = _start_dma(0, xbufs[0], sems[0])
    carry_G = jnp.float32(0.0)
    for t in range(NT):
        nxt = _start_dma(t + 1, xbufs[(t + 1) % 2], sems[(t + 1) % 2]) \
            if t + 1 < NT else None
        (cp, shift) = cur
        cp.wait()
        xb = xbufs[t % 2]

        def _one(g, G, ybase, t=t, xb=xb, shift=shift):
            base = shift + g * 16
            glob = row0_w + t * TILE + g * 16 + lane

            tts = []
            for j in range(16):
                off = jnp.minimum(base + j, TILE - 1) * H
                p = [xb[pl.ds(off + k * 16, 16)] * w1v[k]
                     for k in range(H // 16)]
                while len(p) > 1:
                    p = [p[i] + p[i + 1] for i in range(0, len(p), 2)]
                tts.append(p[0])
            for j in range(16):
                ybuf[pl.ds(ybase + j * 17, 16)] = tts[j]
            acc = plsc.load_gather(ybuf, [l17 + ybase])
            for c in range(1, 16):
                acc = acc + plsc.load_gather(ybuf, [l17 + (ybase + c)])

            cs = G + jnp.cumsum(acc)

            bidx = t * TILE + g * 16 + 1 + lane
            b = plsc.load_gather(batbuf, [bidx])
            b_prev = plsc.load_gather(batbuf, [bidx - 1])
            b_next = plsc.load_gather(batbuf, [bidx + 1])

            vmask = glob < valid_end
            m_s = ((b != b_prev) | (glob == row0_w)) & vmask
            m_e = ((b != b_next) | (glob == valid_end - 1)) & vmask

            plsc.addupdate_scatter(ysum, [b], cs, mask=m_e)
            plsc.addupdate_scatter(ysum, [b], acc - cs, mask=m_s)
            gc = (glob - row0_w + 1).astype(jnp.float32)
            plsc.addupdate_scatter(cnt, [b], gc, mask=m_e)
            plsc.addupdate_scatter(cnt, [b], 1.0 - gc, mask=m_s)
            return cs[15]

        def _pair(p, G, _one=_one):
            G = _one(2 * p, G, 0)
            return _one(2 * p + 1, G, 272)

        carry_G = lax.fori_loop(0, GPT // 2, _pair, carry_G)
        cur = nxt

    pltpu.sync_copy(ysum, py_hbm.at[wid])
    pltpu.sync_copy(cnt, pc_hbm.at[wid])


_sc_partial = functools.partial(
    pl.kernel,
    out_type=(
        jax.ShapeDtypeStruct((NW, S), jnp.float32),
        jax.ShapeDtypeStruct((NW, S), jnp.float32),
    ),
    mesh=plsc.VectorSubcoreMesh(core_axis_name="c", subcore_axis_name="s",
                                num_cores=NC, num_subcores=NS),
    compiler_params=pltpu.CompilerParams(needs_layout_passes=False),
    scratch_types=[
        pltpu.VMEM((TILE * H,), jnp.float32),
        pltpu.VMEM((TILE * H,), jnp.float32),
        pltpu.VMEM((BATBUF,), jnp.int32),
        pltpu.VMEM((H,), jnp.float32),
        pltpu.VMEM((S,), jnp.float32),
        pltpu.VMEM((S,), jnp.float32),
        pltpu.VMEM((2 * 272 + 16,), jnp.float32),
        pltpu.SemaphoreType.DMA,
        pltpu.SemaphoreType.DMA,
    ],
)(_sc_body)


def _combine_body(py_ref, pc_ref, cov_ref, w2_ref, b_ref, out_ref):
    ys = jnp.sum(py_ref[...], axis=0)
    cn = jnp.sum(pc_ref[...], axis=0)
    mean = ys / jnp.maximum(cn, 1.0)
    covw = jnp.dot(cov_ref[...], w2_ref[...],
                   preferred_element_type=jnp.float32)
    out_ref[...] = mean[:, None] + covw + b_ref[0, 0]


_combine = pl.pallas_call(
    _combine_body,
    out_shape=jax.ShapeDtypeStruct((S, 1), jnp.float32),
)


def kernel(x, edge_index, batch, covariates_embedding, W, b):
    del edge_index
    batch32 = batch.astype(jnp.int32)
    pad = (NW - 1) * CH + BATBUF - (N + 1) + 16
    batp = jnp.concatenate(
        [jnp.zeros((1,), jnp.int32), batch32, jnp.zeros((pad,), jnp.int32)])
    w1 = W[0, :H]
    w2 = W[0, H:].reshape(H, 1)
    py, pc = _sc_partial(x.reshape(-1), batp, w1)
    return _combine(py, pc, covariates_embedding, w2, b.reshape(1, 1))
